# baseline (speedup 1.0000x reference)
import functools

import jax
import jax.numpy as jnp
from jax.experimental import pallas as pl
from jax.experimental.pallas import tpu as pltpu

_CLASSES = 80
_CONF = 0.05
_IOU = 0.5
_MAXDET = 100
_PRE = 256
_PBLK = 128


def _nms_block(y1_ref, x1_ref, y2_ref, x2_ref, sc_ref, keep_ref, sup_ref):
    y1 = y1_ref[...]
    x1 = x1_ref[...]
    y2 = y2_ref[...]
    x2 = x2_ref[...]
    sc = sc_ref[...]
    area = (y2 - y1) * (x2 - x1)
    valid = sc > _CONF
    sup_ref[...] = jnp.where(valid, 0.0, 1.0)
    row_id = jax.lax.broadcasted_iota(jnp.int32, (_PRE, _PBLK), 0)

    def body(i, _):
        y1i = y1_ref[pl.ds(i, 1), :]
        x1i = x1_ref[pl.ds(i, 1), :]
        y2i = y2_ref[pl.ds(i, 1), :]
        x2i = x2_ref[pl.ds(i, 1), :]
        supi = sup_ref[pl.ds(i, 1), :]
        area_i = (y2i - y1i) * (x2i - x1i)
        iy1 = jnp.maximum(y1i, y1)
        ix1 = jnp.maximum(x1i, x1)
        iy2 = jnp.minimum(y2i, y2)
        ix2 = jnp.minimum(x2i, x2)
        inter = jnp.maximum(iy2 - iy1, 0.0) * jnp.maximum(ix2 - ix1, 0.0)
        union = area_i + area - inter
        over = inter > _IOU * jnp.maximum(union, 1e-8)
        row = jnp.where(over & (row_id > i), 1.0, 0.0)
        sup = sup_ref[...]
        sup_ref[...] = jnp.maximum(sup, row * (1.0 - supi))
        return 0

    jax.lax.fori_loop(0, _PRE, body, 0, unroll=False)
    keep_ref[...] = jnp.where(valid & (sup_ref[...] < 0.5), sc, -1.0)


def _run_nms(y1t, x1t, y2t, x2t, sct):
    bp = y1t.shape[1]
    grid = (bp // _PBLK,)
    spec = pl.BlockSpec((_PRE, _PBLK), lambda i: (0, i))
    return pl.pallas_call(
        _nms_block,
        grid=grid,
        in_specs=[spec] * 5,
        out_specs=spec,
        out_shape=jax.ShapeDtypeStruct((_PRE, bp), jnp.float32),
        scratch_shapes=[pltpu.VMEM((_PRE, _PBLK), jnp.float32)],
    )(y1t, x1t, y2t, x2t, sct)


def kernel(predictions):
    B, N, _ = predictions.shape
    x1 = predictions[..., 0]
    y1 = predictions[..., 1]
    x2 = predictions[..., 2]
    y2 = predictions[..., 3]
    cls = predictions[..., 4].astype(jnp.int32)
    score = predictions[..., 5]

    cls_scores = jnp.where(
        cls[:, None, :] == jnp.arange(_CLASSES, dtype=jnp.int32)[None, :, None],
        score[:, None, :],
        0.0,
    )
    npad = (-N) % 512
    nchunk = (N + npad) // 512
    cs = jnp.pad(cls_scores, ((0, 0), (0, 0), (0, npad))).reshape(
        B, _CLASSES, nchunk, 512
    )
    s1, i1 = jax.lax.top_k(cs, _PRE)
    g1 = i1 + (jnp.arange(nchunk, dtype=jnp.int32) * 512)[None, None, :, None]
    s2, i2 = jax.lax.top_k(s1.reshape(B, _CLASSES, nchunk * _PRE), _PRE)
    top_scores = s2
    top_idx = jnp.take_along_axis(g1.reshape(B, _CLASSES, nchunk * _PRE), i2, axis=2)

    def _g(src):
        return jax.vmap(lambda s, ix: s[ix])(src, top_idx)

    ty1 = _g(y1)
    tx1 = _g(x1)
    ty2 = _g(y2)
    tx2 = _g(x2)

    bp = B * _CLASSES
    keep_t = _run_nms(
        ty1.reshape(bp, _PRE).T,
        tx1.reshape(bp, _PRE).T,
        ty2.reshape(bp, _PRE).T,
        tx2.reshape(bp, _PRE).T,
        top_scores.reshape(bp, _PRE).T,
    )
    keep = keep_t.T.reshape(B, _CLASSES * _PRE)

    fin_scores, fin_idx = jax.lax.top_k(keep, _MAXDET)
    flat = jnp.stack(
        [
            tx1.reshape(B, -1),
            ty1.reshape(B, -1),
            tx2.reshape(B, -1),
            ty2.reshape(B, -1),
        ],
        axis=-1,
    )
    fin_boxes = jnp.take_along_axis(flat, fin_idx[..., None], axis=1)
    fin_classes = (fin_idx // _PRE).astype(jnp.float32)
    valid_mask = fin_scores > _CONF
    valid_detections = valid_mask.sum(axis=1).astype(jnp.int32)
    out = jnp.concatenate(
        [
            fin_boxes,
            fin_classes[..., None],
            jnp.maximum(fin_scores, 0.0)[..., None],
        ],
        axis=-1,
    )
    out = jnp.where(valid_mask[..., None], out, 0.0)
    return out, valid_detections


# speedup vs baseline: 32.0520x; 32.0520x over previous
import functools

import jax
import jax.numpy as jnp
from jax.experimental import pallas as pl
from jax.experimental.pallas import tpu as pltpu

_CLASSES = 80
_CONF = 0.05
_IOU = 0.5
_MAXDET = 100
_PRE = 256
_PBLK = 128


def _nms_block(y1_ref, x1_ref, y2_ref, x2_ref, sc_ref, keep_ref, sup_ref):
    y1 = y1_ref[...]
    x1 = x1_ref[...]
    y2 = y2_ref[...]
    x2 = x2_ref[...]
    sc = sc_ref[...]
    area = (y2 - y1) * (x2 - x1)
    valid = sc > _CONF
    sup_ref[...] = jnp.where(valid, 0.0, 1.0)
    row_id = jax.lax.broadcasted_iota(jnp.int32, (_PRE, _PBLK), 0)

    def body(i, _):
        y1i = y1_ref[pl.ds(i, 1), :]
        x1i = x1_ref[pl.ds(i, 1), :]
        y2i = y2_ref[pl.ds(i, 1), :]
        x2i = x2_ref[pl.ds(i, 1), :]
        supi = sup_ref[pl.ds(i, 1), :]
        area_i = (y2i - y1i) * (x2i - x1i)
        iy1 = jnp.maximum(y1i, y1)
        ix1 = jnp.maximum(x1i, x1)
        iy2 = jnp.minimum(y2i, y2)
        ix2 = jnp.minimum(x2i, x2)
        inter = jnp.maximum(iy2 - iy1, 0.0) * jnp.maximum(ix2 - ix1, 0.0)
        union = area_i + area - inter
        over = inter > _IOU * jnp.maximum(union, 1e-8)
        row = jnp.where(over & (row_id > i), 1.0, 0.0)
        sup = sup_ref[...]
        sup_ref[...] = jnp.maximum(sup, row * (1.0 - supi))
        return 0

    jax.lax.fori_loop(0, _PRE, body, 0, unroll=False)
    keep_ref[...] = jnp.where(valid & (sup_ref[...] < 0.5), sc, -1.0)


def _run_nms(y1t, x1t, y2t, x2t, sct):
    bp = y1t.shape[1]
    grid = (bp // _PBLK,)
    spec = pl.BlockSpec((_PRE, _PBLK), lambda i: (0, i))
    return pl.pallas_call(
        _nms_block,
        grid=grid,
        in_specs=[spec] * 5,
        out_specs=spec,
        out_shape=jax.ShapeDtypeStruct((_PRE, bp), jnp.float32),
        scratch_shapes=[pltpu.VMEM((_PRE, _PBLK), jnp.float32)],
    )(y1t, x1t, y2t, x2t, sct)


def kernel(predictions):
    B, N, _ = predictions.shape
    x1 = predictions[..., 0]
    y1 = predictions[..., 1]
    x2 = predictions[..., 2]
    y2 = predictions[..., 3]
    cls = predictions[..., 4].astype(jnp.int32)
    score = predictions[..., 5]

    bp = B * _CLASSES
    valid0 = score > _CONF
    cls_key = jnp.where(valid0, cls, 127)
    sbits = jax.lax.bitcast_convert_type(score, jnp.int32)
    neg_sbits = -jnp.where(valid0, sbits, 0)
    idx0 = jnp.broadcast_to(jnp.arange(N, dtype=jnp.int32)[None, :], (B, N))
    s_cls, _, s_idx = jax.lax.sort(
        (cls_key, neg_sbits, idx0), dimension=1, num_keys=3, is_stable=False
    )

    bidx = jnp.broadcast_to(jnp.arange(B, dtype=jnp.int32)[:, None], (B, N))
    cnt = (
        jnp.zeros((B, 128), jnp.int32)
        .at[bidx.reshape(-1), cls_key.reshape(-1)]
        .add(1)
    )[:, :_CLASSES]
    off = jnp.cumsum(cnt, axis=1) - cnt

    r_col = jnp.arange(_PRE, dtype=jnp.int32)[:, None]
    off_f = off.reshape(1, bp)
    cnt_f = cnt.reshape(1, bp)
    slot_ok = r_col < cnt_f
    pos = jnp.where(slot_ok, off_f + r_col, 0)
    bofs = (jnp.arange(bp, dtype=jnp.int32) // _CLASSES * N).reshape(1, bp)
    src = jnp.take(s_idx.reshape(-1), bofs + pos)
    top_scores_t = jnp.where(slot_ok, jnp.take(score.reshape(-1), bofs + src), 0.0)

    def _g(coord):
        return jnp.take(coord.reshape(-1), bofs + src)

    ty1 = _g(y1)
    tx1 = _g(x1)
    ty2 = _g(y2)
    tx2 = _g(x2)

    keep_t = _run_nms(ty1, tx1, ty2, tx2, top_scores_t)
    keep = keep_t.T.reshape(B, _CLASSES * _PRE)

    fin_scores, fin_idx = jax.lax.top_k(keep, _MAXDET)
    fc = fin_idx // _PRE
    fr = fin_idx % _PRE
    tpos = fr * bp + jnp.arange(B, dtype=jnp.int32)[:, None] * _CLASSES + fc
    fin_boxes = jnp.stack(
        [
            jnp.take(tx1.reshape(-1), tpos),
            jnp.take(ty1.reshape(-1), tpos),
            jnp.take(tx2.reshape(-1), tpos),
            jnp.take(ty2.reshape(-1), tpos),
        ],
        axis=-1,
    )
    fin_classes = fc.astype(jnp.float32)
    valid_mask = fin_scores > _CONF
    valid_detections = valid_mask.sum(axis=1).astype(jnp.int32)
    out = jnp.concatenate(
        [
            fin_boxes,
            fin_classes[..., None],
            jnp.maximum(fin_scores, 0.0)[..., None],
        ],
        axis=-1,
    )
    out = jnp.where(valid_mask[..., None], out, 0.0)
    return out, valid_detections
